# baseline (speedup 1.0000x reference)
import functools

import jax
import jax.numpy as jnp
from jax import lax
from jax.experimental import pallas as pl
from jax.experimental.pallas import tpu as pltpu
from jax.experimental.pallas import tpu_sc as plsc

N_NODES = 100000
NDUM = 224
NACC = N_NODES + NDUM
NC = 2
NS = 16
NW = NC * NS
CE = 256
NR = CE // 128
CT = 128
RT = NACC // NS
BN = 2048
NP = 49 * BN

_f32 = jnp.float32
_i32 = jnp.int32


def _sc_body(packed, e0r, e1r, tbr, t1r, t2r, t3r, wpk, z8,
             outa, outb,
             wbuf, ig0a, ig0b, ig1a, ig1b, is0a, is0b,
             r0a, r0b, r1a, r1b, eaa, eab, eba, ebb,
             acc_a, acc_b, semga, semgb, semsa, semsb,
             n_ch_e, n_ch_t):
    c = lax.axis_index("c")
    s = lax.axis_index("s")
    w = c * NS + s
    r0 = s * RT

    ig0 = (ig0a, ig0b)
    ig1 = (ig1a, ig1b)
    is0 = (is0a, is0b)
    rows0 = (r0a, r0b)
    rows1 = (r1a, r1b)
    ea = (eaa, eab)
    eb = (eba, ebb)
    semg = (semga, semgb)
    sems = (semsa, semsb)

    pltpu.sync_copy(z8.at[pl.ds(r0, RT)], acc_a.at[pl.ds(r0, RT)])
    pltpu.sync_copy(z8.at[pl.ds(r0, RT)], acc_b.at[pl.ds(r0, RT)])
    for p in range(2):
        pltpu.sync_copy(z8.at[pl.ds(0, CE)], ea[p])
        pltpu.sync_copy(z8.at[pl.ds(0, CE)], eb[p])
    pltpu.sync_copy(wpk, wbuf)
    plsc.subcore_barrier()

    wv = [wbuf[k, :] for k in range(90)]
    W1S = lambda k, j: wv[k * 3 + j]
    B1S = lambda j: wv[18 + j]
    W2S = lambda j, m: wv[21 + j * 9 + m]
    B2S = lambda m: wv[48 + m]
    W1C = lambda k, j: wv[57 + k * 3 + j]
    B1C = lambda j: wv[75 + j]
    W2C = lambda j, m: wv[78 + j * 3 + m]
    B2C = lambda m: wv[87 + m]

    iota16 = lax.iota(_i32, 16)

    def col(c_):
        return jnp.full((16,), c_, _i32)

    def silu(x):
        return x / (1.0 + jnp.exp(-x))

    erows = n_ch_e * NR

    def fire_gathers(ck, p):
        rowbase = w * erows + ck * NR
        pltpu.sync_copy(e0r.at[pl.ds(rowbase, NR)], ig0[p])
        pltpu.sync_copy(e1r.at[pl.ds(rowbase, NR)], ig1[p])
        for j in range(NR):
            pltpu.async_copy(
                packed.at[ig0[p].at[j]], rows0[p].at[pl.ds(j * 128, 128)], semg[p])
            pltpu.async_copy(
                packed.at[ig1[p].at[j]], rows1[p].at[pl.ds(j * 128, 128)], semg[p])

    def drain_gathers(p):
        for j in range(NR):
            pltpu.make_async_copy(
                packed.at[ig0[p].at[j]], rows0[p].at[pl.ds(j * 128, 128)],
                semg[p]).wait()
            pltpu.make_async_copy(
                packed.at[ig1[p].at[j]], rows1[p].at[pl.ds(j * 128, 128)],
                semg[p]).wait()

    def drain_scatters(p):
        for j in range(NR):
            pltpu.make_async_copy(
                z8.at[pl.ds(0, 128)], acc_a.at[pl.ds(0, 128)], sems[p]).wait()
            pltpu.make_async_copy(
                z8.at[pl.ds(0, 128)], acc_b.at[pl.ds(0, 128)], sems[p]).wait()

    def edge_compute(p):
        @pl.loop(0, CE // 16)
        def _grp(g):
            rid = g * 16 + iota16
            ld0 = lambda k: plsc.load_gather(rows0[p], [rid, col(k)])
            ld1 = lambda k: plsc.load_gather(rows1[p], [rid, col(k)])
            s0 = [ld0(k) for k in range(3)]
            s1 = [ld1(k) for k in range(3)]
            v1 = [ld1(3 + k) for k in range(3)]
            p0 = [ld0(6 + k) for k in range(3)]
            p1 = [ld1(6 + k) for k in range(3)]
            h = s0 + s1
            hid = []
            for j in range(3):
                a = B1S(j)
                for k in range(6):
                    a = a + h[k] * W1S(k, j)
                hid.append(silu(a))
            gate = []
            for m in range(9):
                a = B2S(m)
                for j in range(3):
                    a = a + hid[j] * W2S(j, m)
                gate.append(a)
            for d in range(3):
                plsc.store_scatter(ea[p], [rid, col(d)], gate[6 + d])
                plsc.store_scatter(ea[p], [rid, col(3 + d)], gate[d] * v1[d])
                plsc.store_scatter(eb[p], [rid, col(d)], gate[3 + d] * (p1[d] - p0[d]))

    def edge_step(ck, p, q):
        @pl.when(ck + 1 < n_ch_e)
        def _():
            fire_gathers(ck + 1, q)
        drain_gathers(p)

        @pl.when(ck >= 2)
        def _():
            drain_scatters(p)
        edge_compute(p)
        pltpu.sync_copy(e0r.at[pl.ds(w * erows + ck * NR, NR)], is0[p])
        for j in range(NR):
            pltpu.async_copy(
                ea[p].at[pl.ds(j * 128, 128)], acc_a.at[is0[p].at[j]],
                sems[p], add=True)
            pltpu.async_copy(
                eb[p].at[pl.ds(j * 128, 128)], acc_b.at[is0[p].at[j]],
                sems[p], add=True)

    fire_gathers(0, 0)

    @pl.loop(0, n_ch_e // 2)
    def _edge_iter(k2):
        edge_step(k2 * 2, 0, 1)
        edge_step(k2 * 2 + 1, 1, 0)

    drain_scatters(0)
    drain_scatters(1)

    pltpu.sync_copy(z8.at[pl.ds(0, CE)], ea[0])

    @pl.loop(0, n_ch_t)
    def _tri_chunk(i):
        rowbase = w * n_ch_t + i
        pltpu.sync_copy(tbr.at[pl.ds(rowbase, 1)], ig0[0].at[pl.ds(0, 1)])
        pltpu.sync_copy(t1r.at[pl.ds(rowbase, 1)], ig0[0].at[pl.ds(1, 1)])
        pltpu.sync_copy(t2r.at[pl.ds(rowbase, 1)], ig1[0].at[pl.ds(0, 1)])
        pltpu.sync_copy(t3r.at[pl.ds(rowbase, 1)], ig1[0].at[pl.ds(1, 1)])
        ds = [
            pltpu.async_copy(packed.at[ig0[0].at[0]],
                             rows0[0].at[pl.ds(0, 128)], semg[0]),
            pltpu.async_copy(packed.at[ig0[0].at[1]],
                             rows0[0].at[pl.ds(128, 128)], semg[0]),
            pltpu.async_copy(packed.at[ig1[0].at[0]],
                             rows1[0].at[pl.ds(0, 128)], semg[0]),
            pltpu.async_copy(packed.at[ig1[0].at[1]],
                             rows1[0].at[pl.ds(128, 128)], semg[0]),
        ]
        for d in ds:
            d.wait()

        @pl.loop(0, CT // 16)
        def _grp(g):
            rid = g * 16 + iota16
            ldb = lambda k: plsc.load_gather(rows0[0], [rid, col(k)])
            ld1 = lambda k: plsc.load_gather(rows0[0], [rid + CT, col(k)])
            ld2 = lambda k: plsc.load_gather(rows1[0], [rid, col(k)])
            ld3 = lambda k: plsc.load_gather(rows1[0], [rid + CT, col(k)])
            cb = [ldb(9 + k) for k in range(3)]
            c1 = [ld1(9 + k) for k in range(3)]
            c2 = [ld2(9 + k) for k in range(3)]
            c3 = [ld3(9 + k) for k in range(3)]
            pb = [ldb(6 + k) for k in range(3)]
            q1 = [ld1(6 + k) for k in range(3)]
            q2 = [ld2(6 + k) for k in range(3)]
            q3 = [ld3(6 + k) for k in range(3)]

            gate = [B2C(m) * 3.0 for m in range(3)]
            for ct in (c1, c2, c3):
                hid = []
                for j in range(3):
                    a = B1C(j)
                    for k in range(3):
                        a = a + cb[k] * W1C(k, j)
                    for k in range(3):
                        a = a + ct[k] * W1C(3 + k, j)
                    hid.append(silu(a))
                for m in range(3):
                    acc = gate[m]
                    for j in range(3):
                        acc = acc + hid[j] * W2C(j, m)
                    gate[m] = acc

            r1 = [pb[k] - q1[k] for k in range(3)]
            r2 = [pb[k] - q2[k] for k in range(3)]
            r3 = [pb[k] - q3[k] for k in range(3)]
            cx = r2[1] * r3[2] - r2[2] * r3[1]
            cy = r2[2] * r3[0] - r2[0] * r3[2]
            cz = r2[0] * r3[1] - r2[1] * r3[0]
            stp = r1[0] * cx + r1[1] * cy + r1[2] * cz
            inv = 1.0 / (stp + 0.01)
            for m in range(3):
                plsc.store_scatter(ea[0], [rid, col(3 + m)], gate[m] * inv)

        d = pltpu.async_copy(ea[0].at[pl.ds(0, 128)], acc_b.at[ig0[0].at[0]],
                             sems[0], add=True)
        d.wait()

    plsc.subcore_barrier()
    pltpu.sync_copy(acc_a.at[pl.ds(r0, RT)], outa.at[pl.ds(c * NACC + r0, RT)])
    pltpu.sync_copy(acc_b.at[pl.ds(r0, RT)], outb.at[pl.ds(c * NACC + r0, RT)])


def _tc_body(wv_ref, bv_ref, s_ref, c_ref, v_ref, ss_ref, vv_ref, ev_ref, ch_ref,
             os_ref, oc_ref, ov_ref):
    sc = s_ref[...]
    ch = c_ref[...]
    ve = v_ref[...]
    m_ss = ss_ref[0] + ss_ref[1]
    m_vv = vv_ref[0] + vv_ref[1]
    m_ev = ev_ref[0] + ev_ref[1]
    m_ch = ch_ref[0] + ch_ref[1]
    norm2 = jnp.zeros((BN, 1), _f32)
    for j in range(3):
        vvj = (bv_ref[j] + ve[:, 0:1] * wv_ref[0, j] + ve[:, 1:2] * wv_ref[1, j]
               + ve[:, 2:3] * wv_ref[2, j])
        norm2 = norm2 + vvj * vvj
    nrm = jnp.sqrt(norm2)
    os_ref[...] = sc + m_ss + sc * nrm
    oc_ref[...] = ch + m_ch
    ov_ref[...] = ve + m_vv + m_ev


def kernel(node_scalar, node_chiral, node_vector, edge_index, triplet_index, pos,
           W1s, b1s, W2s, b2s, W1c, b1c, W2c, b2c, WV, bV):
    E = edge_index.shape[0]
    T = triplet_index.shape[0]
    n_ch_e = -(-E // (NW * CE))
    n_ch_e += n_ch_e % 2
    n_ch_t = -(-T // (NW * CT))
    EPAD = NW * CE * n_ch_e
    TPAD = NW * CT * n_ch_t

    zcol = jnp.zeros((NACC - N_NODES, 3), _f32)
    packed = jnp.concatenate([
        jnp.concatenate([node_scalar, zcol], 0),
        jnp.concatenate([node_vector, zcol], 0),
        jnp.concatenate([pos, zcol], 0),
        jnp.concatenate([node_chiral, zcol], 0),
        jnp.zeros((NACC, 4), _f32),
    ], axis=1)

    def pad_idx(x, total):
        npad = total - x.shape[0]
        fill = N_NODES + (jnp.arange(npad, dtype=_i32) % NDUM)
        return jnp.concatenate([x.astype(_i32), fill]).reshape(-1, 128)

    e0r = pad_idx(edge_index[:, 0], EPAD)
    e1r = pad_idx(edge_index[:, 1], EPAD)
    tbr = pad_idx(triplet_index[:, 0], TPAD)
    t1r = pad_idx(triplet_index[:, 1], TPAD)
    t2r = pad_idx(triplet_index[:, 2], TPAD)
    t3r = pad_idx(triplet_index[:, 3], TPAD)

    wpk = jnp.concatenate([
        W1s.reshape(-1), b1s, W2s.reshape(-1), b2s,
        W1c.reshape(-1), b1c, W2c.reshape(-1), b2c,
    ]).astype(_f32)[:, None] * jnp.ones((1, 16), _f32)

    z8 = jnp.zeros((NACC, 8), _f32)

    mesh = plsc.VectorSubcoreMesh(core_axis_name="c", subcore_axis_name="s")
    sc_fn = functools.partial(
        pl.kernel,
        out_type=(jax.ShapeDtypeStruct((NC * NACC, 8), _f32),
                  jax.ShapeDtypeStruct((NC * NACC, 8), _f32)),
        mesh=mesh,
        compiler_params=pltpu.CompilerParams(
            needs_layout_passes=False, use_tc_tiling_on_sc=False),
        scratch_types=[
            pltpu.VMEM((90, 16), _f32),
            pltpu.VMEM((NR, 128), _i32),
            pltpu.VMEM((NR, 128), _i32),
            pltpu.VMEM((NR, 128), _i32),
            pltpu.VMEM((NR, 128), _i32),
            pltpu.VMEM((NR, 128), _i32),
            pltpu.VMEM((NR, 128), _i32),
            pltpu.VMEM((CE, 16), _f32),
            pltpu.VMEM((CE, 16), _f32),
            pltpu.VMEM((CE, 16), _f32),
            pltpu.VMEM((CE, 16), _f32),
            pltpu.VMEM((CE, 8), _f32),
            pltpu.VMEM((CE, 8), _f32),
            pltpu.VMEM((CE, 8), _f32),
            pltpu.VMEM((CE, 8), _f32),
            pltpu.VMEM_SHARED((NACC, 8), _f32),
            pltpu.VMEM_SHARED((NACC, 8), _f32),
            pltpu.SemaphoreType.DMA,
            pltpu.SemaphoreType.DMA,
            pltpu.SemaphoreType.DMA,
            pltpu.SemaphoreType.DMA,
        ],
    )(functools.partial(_sc_body, n_ch_e=n_ch_e, n_ch_t=n_ch_t))

    outa, outb = sc_fn(packed, e0r, e1r, tbr, t1r, t2r, t3r, wpk, z8)

    ea = outa.reshape(NC, NACC, 8)
    eb = outb.reshape(NC, NACC, 8)

    def prep(x):
        return jnp.pad(x[:, :N_NODES, :], ((0, 0), (0, NP - N_NODES), (0, 0)))

    ssP = prep(ea[:, :, 0:3])
    vvP = prep(ea[:, :, 3:6])
    evP = prep(eb[:, :, 0:3])
    chP = prep(eb[:, :, 3:6])

    def padn(x):
        return jnp.pad(x, ((0, NP - N_NODES), (0, 0)))

    sP = padn(node_scalar)
    cP = padn(node_chiral)
    vP = padn(node_vector)

    nblk = pl.BlockSpec((BN, 3), lambda i: (i, 0))
    pblk = pl.BlockSpec((NC, BN, 3), lambda i: (0, i, 0))
    smem = pl.BlockSpec(memory_space=pltpu.SMEM)
    outs = pl.pallas_call(
        _tc_body,
        grid=(NP // BN,),
        in_specs=[smem, smem, nblk, nblk, nblk, pblk, pblk, pblk, pblk],
        out_specs=[nblk, nblk, nblk],
        out_shape=[jax.ShapeDtypeStruct((NP, 3), _f32)] * 3,
    )(WV, bV, sP, cP, vP, ssP, vvP, evP, chP)

    return (outs[0][:N_NODES], outs[1][:N_NODES], outs[2][:N_NODES])


# speedup vs baseline: 1.4271x; 1.4271x over previous
import functools

import jax
import jax.numpy as jnp
from jax import lax
from jax.experimental import pallas as pl
from jax.experimental.pallas import tpu as pltpu
from jax.experimental.pallas import tpu_sc as plsc

N_NODES = 100000
NDUM = 224
NACC = N_NODES + NDUM
NC = 2
NS = 16
NW = NC * NS
CE = 256
NR = CE // 128
CT = 256
NRT = CT // 128
RT = NACC // NS
BN = 2048
NP = 49 * BN

_f32 = jnp.float32
_i32 = jnp.int32


def _sc_body(packed, ecomb, tcomb, wpk, z8,
             outa, outb,
             wbuf, igca, igcb, is0, is1, is2, is3, itc,
             r0a, r0b, r1a, r1b, eaa, eab, eba, ebb,
             acc_a, acc_b, semga, semgb, semsa, semsb,
             semi0, semi1, semi2, semi3,
             n_ch_e, n_ch_t):
    c = lax.axis_index("c")
    s = lax.axis_index("s")
    w = c * NS + s
    r0 = s * RT

    igc = (igca, igcb)
    iss = (is0, is1, is2, is3)
    rows0 = (r0a, r0b)
    rows1 = (r1a, r1b)
    ea = (eaa, eab)
    eb = (eba, ebb)
    semg = (semga, semgb)
    sems = (semsa, semsb)
    semi = (semi0, semi1, semi2, semi3)

    pltpu.sync_copy(z8.at[pl.ds(r0, RT)], acc_a.at[pl.ds(r0, RT)])
    pltpu.sync_copy(z8.at[pl.ds(r0, RT)], acc_b.at[pl.ds(r0, RT)])
    for p in range(2):
        pltpu.sync_copy(z8.at[pl.ds(0, CE)], ea[p])
        pltpu.sync_copy(z8.at[pl.ds(0, CE)], eb[p])
    pltpu.sync_copy(wpk, wbuf)
    plsc.subcore_barrier()

    wv = [wbuf[k, :] for k in range(90)]
    W1S = lambda k, j: wv[k * 3 + j]
    B1S = lambda j: wv[18 + j]
    W2S = lambda j, m: wv[21 + j * 9 + m]
    B2S = lambda m: wv[48 + m]
    W1C = lambda k, j: wv[57 + k * 3 + j]
    B1C = lambda j: wv[75 + j]
    W2C = lambda j, m: wv[78 + j * 3 + m]
    B2C = lambda m: wv[87 + m]

    iota16 = lax.iota(_i32, 16)

    def col(c_):
        return jnp.full((16,), c_, _i32)

    def silu(x):
        return x / (1.0 + jnp.exp(-x))

    def fire_gathers(ck, p, isl):
        base = (w * n_ch_e + ck) * 2 * NR
        pltpu.sync_copy(ecomb.at[pl.ds(base, 2 * NR)], igc[p])
        pltpu.async_copy(ecomb.at[pl.ds(base, NR)], iss[isl], semi[isl])
        for j in range(NR):
            pltpu.async_copy(
                packed.at[igc[p].at[j]], rows0[p].at[pl.ds(j * 128, 128)], semg[p])
            pltpu.async_copy(
                packed.at[igc[p].at[NR + j]], rows1[p].at[pl.ds(j * 128, 128)],
                semg[p])

    def drain_gathers(p):
        for j in range(2 * NR):
            pltpu.make_async_copy(
                packed.at[igc[p].at[0]], rows0[p].at[pl.ds(0, 128)],
                semg[p]).wait()

    def drain_scatters(p):
        for j in range(2 * NR):
            pltpu.make_async_copy(
                z8.at[pl.ds(0, 128)], acc_a.at[pl.ds(0, 128)], sems[p]).wait()

    def edge_compute(p):
        @pl.loop(0, CE // 16)
        def _grp(g):
            rid = g * 16 + iota16
            ld0 = lambda k: plsc.load_gather(rows0[p], [rid, col(k)])
            ld1 = lambda k: plsc.load_gather(rows1[p], [rid, col(k)])
            s0 = [ld0(k) for k in range(3)]
            s1 = [ld1(k) for k in range(3)]
            v1 = [ld1(3 + k) for k in range(3)]
            p0 = [ld0(6 + k) for k in range(3)]
            p1 = [ld1(6 + k) for k in range(3)]
            h = s0 + s1
            hid = []
            for j in range(3):
                a = B1S(j)
                for k in range(6):
                    a = a + h[k] * W1S(k, j)
                hid.append(silu(a))
            gate = []
            for m in range(9):
                a = B2S(m)
                for j in range(3):
                    a = a + hid[j] * W2S(j, m)
                gate.append(a)
            for d in range(3):
                plsc.store_scatter(ea[p], [rid, col(d)], gate[6 + d])
                plsc.store_scatter(ea[p], [rid, col(3 + d)], gate[d] * v1[d])
                plsc.store_scatter(eb[p], [rid, col(d)], gate[3 + d] * (p1[d] - p0[d]))

    def edge_step(ck, p, q, isl, isl_next):
        @pl.when(ck + 1 < n_ch_e)
        def _():
            fire_gathers(ck + 1, q, isl_next)
        drain_gathers(p)

        @pl.when(ck >= 2)
        def _():
            drain_scatters(p)
        edge_compute(p)
        pltpu.make_async_copy(
            ecomb.at[pl.ds(0, NR)], iss[isl], semi[isl]).wait()
        for j in range(NR):
            pltpu.async_copy(
                ea[p].at[pl.ds(j * 128, 128)], acc_a.at[iss[isl].at[j]],
                sems[p], add=True)
            pltpu.async_copy(
                eb[p].at[pl.ds(j * 128, 128)], acc_b.at[iss[isl].at[j]],
                sems[p], add=True)

    fire_gathers(0, 0, 0)

    @pl.loop(0, n_ch_e // 4)
    def _edge_iter(k4):
        for u in range(4):
            edge_step(k4 * 4 + u, u % 2, 1 - u % 2, u, (u + 1) % 4)

    drain_scatters(0)
    drain_scatters(1)

    pltpu.sync_copy(z8.at[pl.ds(0, CE)], ea[0])

    @pl.loop(0, n_ch_t)
    def _tri_chunk(i):
        base = (w * n_ch_t + i) * 4 * NRT
        pltpu.sync_copy(tcomb.at[pl.ds(base, 4 * NRT)], itc)
        ds = []
        for j in range(NRT):
            ds.append(pltpu.async_copy(
                packed.at[itc.at[j]], rows0[0].at[pl.ds(j * 128, 128)], semg[0]))
            ds.append(pltpu.async_copy(
                packed.at[itc.at[NRT + j]], rows0[1].at[pl.ds(j * 128, 128)],
                semg[0]))
            ds.append(pltpu.async_copy(
                packed.at[itc.at[2 * NRT + j]], rows1[0].at[pl.ds(j * 128, 128)],
                semg[0]))
            ds.append(pltpu.async_copy(
                packed.at[itc.at[3 * NRT + j]], rows1[1].at[pl.ds(j * 128, 128)],
                semg[0]))
        for d in ds:
            d.wait()

        @pl.loop(0, CT // 16)
        def _grp(g):
            rid = g * 16 + iota16
            ldb = lambda k: plsc.load_gather(rows0[0], [rid, col(k)])
            ld1 = lambda k: plsc.load_gather(rows0[1], [rid, col(k)])
            ld2 = lambda k: plsc.load_gather(rows1[0], [rid, col(k)])
            ld3 = lambda k: plsc.load_gather(rows1[1], [rid, col(k)])
            cb = [ldb(9 + k) for k in range(3)]
            c1 = [ld1(9 + k) for k in range(3)]
            c2 = [ld2(9 + k) for k in range(3)]
            c3 = [ld3(9 + k) for k in range(3)]
            pb = [ldb(6 + k) for k in range(3)]
            q1 = [ld1(6 + k) for k in range(3)]
            q2 = [ld2(6 + k) for k in range(3)]
            q3 = [ld3(6 + k) for k in range(3)]

            gate = [B2C(m) * 3.0 for m in range(3)]
            for ct in (c1, c2, c3):
                hid = []
                for j in range(3):
                    a = B1C(j)
                    for k in range(3):
                        a = a + cb[k] * W1C(k, j)
                    for k in range(3):
                        a = a + ct[k] * W1C(3 + k, j)
                    hid.append(silu(a))
                for m in range(3):
                    acc = gate[m]
                    for j in range(3):
                        acc = acc + hid[j] * W2C(j, m)
                    gate[m] = acc

            r1 = [pb[k] - q1[k] for k in range(3)]
            r2 = [pb[k] - q2[k] for k in range(3)]
            r3 = [pb[k] - q3[k] for k in range(3)]
            cx = r2[1] * r3[2] - r2[2] * r3[1]
            cy = r2[2] * r3[0] - r2[0] * r3[2]
            cz = r2[0] * r3[1] - r2[1] * r3[0]
            stp = r1[0] * cx + r1[1] * cy + r1[2] * cz
            inv = 1.0 / (stp + 0.01)
            for m in range(3):
                plsc.store_scatter(ea[0], [rid, col(3 + m)], gate[m] * inv)

        ds2 = []
        for j in range(NRT):
            ds2.append(pltpu.async_copy(
                ea[0].at[pl.ds(j * 128, 128)], acc_b.at[itc.at[j]],
                sems[0], add=True))
        for d in ds2:
            d.wait()

    plsc.subcore_barrier()
    pltpu.sync_copy(acc_a.at[pl.ds(r0, RT)], outa.at[pl.ds(c * NACC + r0, RT)])
    pltpu.sync_copy(acc_b.at[pl.ds(r0, RT)], outb.at[pl.ds(c * NACC + r0, RT)])


def _tc_body(wv_ref, bv_ref, s_ref, c_ref, v_ref, ss_ref, vv_ref, ev_ref, ch_ref,
             os_ref, oc_ref, ov_ref):
    sc = s_ref[...]
    ch = c_ref[...]
    ve = v_ref[...]
    m_ss = ss_ref[0] + ss_ref[1]
    m_vv = vv_ref[0] + vv_ref[1]
    m_ev = ev_ref[0] + ev_ref[1]
    m_ch = ch_ref[0] + ch_ref[1]
    norm2 = jnp.zeros((BN, 1), _f32)
    for j in range(3):
        vvj = (bv_ref[j] + ve[:, 0:1] * wv_ref[0, j] + ve[:, 1:2] * wv_ref[1, j]
               + ve[:, 2:3] * wv_ref[2, j])
        norm2 = norm2 + vvj * vvj
    nrm = jnp.sqrt(norm2)
    os_ref[...] = sc + m_ss + sc * nrm
    oc_ref[...] = ch + m_ch
    ov_ref[...] = ve + m_vv + m_ev


def kernel(node_scalar, node_chiral, node_vector, edge_index, triplet_index, pos,
           W1s, b1s, W2s, b2s, W1c, b1c, W2c, b2c, WV, bV):
    E = edge_index.shape[0]
    T = triplet_index.shape[0]
    n_ch_e = -(-E // (NW * CE))
    n_ch_e += (-n_ch_e) % 4
    n_ch_t = -(-T // (NW * CT))
    EPAD = NW * CE * n_ch_e
    TPAD = NW * CT * n_ch_t

    zcol = jnp.zeros((NACC - N_NODES, 3), _f32)
    packed = jnp.concatenate([
        jnp.concatenate([node_scalar, zcol], 0),
        jnp.concatenate([node_vector, zcol], 0),
        jnp.concatenate([pos, zcol], 0),
        jnp.concatenate([node_chiral, zcol], 0),
        jnp.zeros((NACC, 4), _f32),
    ], axis=1)

    def pad_idx(x, total, nr):
        npad = total - x.shape[0]
        fill = N_NODES + (jnp.arange(npad, dtype=_i32) % NDUM)
        return jnp.concatenate([x.astype(_i32), fill]).reshape(-1, nr, 128)

    ecomb = jnp.concatenate([
        pad_idx(edge_index[:, 0], EPAD, NR),
        pad_idx(edge_index[:, 1], EPAD, NR),
    ], axis=1).reshape(-1, 128)
    tcomb = jnp.concatenate([
        pad_idx(triplet_index[:, 0], TPAD, NRT),
        pad_idx(triplet_index[:, 1], TPAD, NRT),
        pad_idx(triplet_index[:, 2], TPAD, NRT),
        pad_idx(triplet_index[:, 3], TPAD, NRT),
    ], axis=1).reshape(-1, 128)

    wpk = jnp.concatenate([
        W1s.reshape(-1), b1s, W2s.reshape(-1), b2s,
        W1c.reshape(-1), b1c, W2c.reshape(-1), b2c,
    ]).astype(_f32)[:, None] * jnp.ones((1, 16), _f32)

    z8 = jnp.zeros((NACC, 8), _f32)

    mesh = plsc.VectorSubcoreMesh(core_axis_name="c", subcore_axis_name="s")
    sc_fn = functools.partial(
        pl.kernel,
        out_type=(jax.ShapeDtypeStruct((NC * NACC, 8), _f32),
                  jax.ShapeDtypeStruct((NC * NACC, 8), _f32)),
        mesh=mesh,
        compiler_params=pltpu.CompilerParams(
            needs_layout_passes=False, use_tc_tiling_on_sc=False),
        scratch_types=[
            pltpu.VMEM((90, 16), _f32),
            pltpu.VMEM((2 * NR, 128), _i32),
            pltpu.VMEM((2 * NR, 128), _i32),
            pltpu.VMEM((NR, 128), _i32),
            pltpu.VMEM((NR, 128), _i32),
            pltpu.VMEM((NR, 128), _i32),
            pltpu.VMEM((NR, 128), _i32),
            pltpu.VMEM((4 * NRT, 128), _i32),
            pltpu.VMEM((CE, 16), _f32),
            pltpu.VMEM((CE, 16), _f32),
            pltpu.VMEM((CE, 16), _f32),
            pltpu.VMEM((CE, 16), _f32),
            pltpu.VMEM((CE, 8), _f32),
            pltpu.VMEM((CE, 8), _f32),
            pltpu.VMEM((CE, 8), _f32),
            pltpu.VMEM((CE, 8), _f32),
            pltpu.VMEM_SHARED((NACC, 8), _f32),
            pltpu.VMEM_SHARED((NACC, 8), _f32),
            pltpu.SemaphoreType.DMA,
            pltpu.SemaphoreType.DMA,
            pltpu.SemaphoreType.DMA,
            pltpu.SemaphoreType.DMA,
            pltpu.SemaphoreType.DMA,
            pltpu.SemaphoreType.DMA,
            pltpu.SemaphoreType.DMA,
            pltpu.SemaphoreType.DMA,
        ],
    )(functools.partial(_sc_body, n_ch_e=n_ch_e, n_ch_t=n_ch_t))

    outa, outb = sc_fn(packed, ecomb, tcomb, wpk, z8)

    ea = outa.reshape(NC, NACC, 8)
    eb = outb.reshape(NC, NACC, 8)

    def prep(x):
        return jnp.pad(x[:, :N_NODES, :], ((0, 0), (0, NP - N_NODES), (0, 0)))

    ssP = prep(ea[:, :, 0:3])
    vvP = prep(ea[:, :, 3:6])
    evP = prep(eb[:, :, 0:3])
    chP = prep(eb[:, :, 3:6])

    def padn(x):
        return jnp.pad(x, ((0, NP - N_NODES), (0, 0)))

    sP = padn(node_scalar)
    cP = padn(node_chiral)
    vP = padn(node_vector)

    nblk = pl.BlockSpec((BN, 3), lambda i: (i, 0))
    pblk = pl.BlockSpec((NC, BN, 3), lambda i: (0, i, 0))
    smem = pl.BlockSpec(memory_space=pltpu.SMEM)
    outs = pl.pallas_call(
        _tc_body,
        grid=(NP // BN,),
        in_specs=[smem, smem, nblk, nblk, nblk, pblk, pblk, pblk, pblk],
        out_specs=[nblk, nblk, nblk],
        out_shape=[jax.ShapeDtypeStruct((NP, 3), _f32)] * 3,
    )(WV, bV, sP, cP, vP, ssP, vvP, evP, chP)

    return (outs[0][:N_NODES], outs[1][:N_NODES], outs[2][:N_NODES])


# speedup vs baseline: 1.5339x; 1.0748x over previous
import functools

import jax
import jax.numpy as jnp
from jax import lax
from jax.experimental import pallas as pl
from jax.experimental.pallas import tpu as pltpu
from jax.experimental.pallas import tpu_sc as plsc

N_NODES = 100000
NDUM = 224
NACC = N_NODES + NDUM
NC = 2
NS = 16
NW = NC * NS
CE = 256
NR = CE // 128
CT = 128
NRT = CT // 128
RT = NACC // NS
BN = 2048
NP = 49 * BN

_f32 = jnp.float32
_i32 = jnp.int32


def _sc_body(packed, ecomb, tcomb, wpk, z8,
             outa, outb,
             wbuf, igca, igcb, is0, is1, is2, is3,
             r0a, r0b, r1a, r1b, eaa, eab, eba, ebb,
             acc_a, acc_b, semga, semgb, semsa, semsb,
             semi0, semi1, semi2, semi3,
             n_ch_e, n_ch_t):
    c = lax.axis_index("c")
    s = lax.axis_index("s")
    w = c * NS + s
    r0 = s * RT

    igc = (igca, igcb)
    iss = (is0, is1, is2, is3)
    rows0 = (r0a, r0b)
    rows1 = (r1a, r1b)
    ea = (eaa, eab)
    eb = (eba, ebb)
    semg = (semga, semgb)
    sems = (semsa, semsb)
    semi = (semi0, semi1, semi2, semi3)

    pltpu.sync_copy(z8.at[pl.ds(r0, RT)], acc_a.at[pl.ds(r0, RT)])
    pltpu.sync_copy(z8.at[pl.ds(r0, RT)], acc_b.at[pl.ds(r0, RT)])
    for p in range(2):
        pltpu.sync_copy(z8.at[pl.ds(0, CE)], ea[p])
        pltpu.sync_copy(z8.at[pl.ds(0, CE)], eb[p])
    pltpu.sync_copy(wpk, wbuf)
    plsc.subcore_barrier()

    wv = [wbuf[k, :] for k in range(90)]
    W1S = lambda k, j: wv[k * 3 + j]
    B1S = lambda j: wv[18 + j]
    W2S = lambda j, m: wv[21 + j * 9 + m]
    B2S = lambda m: wv[48 + m]
    W1C = lambda k, j: wv[57 + k * 3 + j]
    B1C = lambda j: wv[75 + j]
    W2C = lambda j, m: wv[78 + j * 3 + m]
    B2C = lambda m: wv[87 + m]

    iota16 = lax.iota(_i32, 16)

    def col(c_):
        return jnp.full((16,), c_, _i32)

    def silu(x):
        return x / (1.0 + jnp.exp(-x))

    def fire_gathers(ck, p, isl):
        base = (w * n_ch_e + ck) * 2 * NR
        pltpu.sync_copy(ecomb.at[pl.ds(base, 2 * NR)], igc[p])
        pltpu.async_copy(ecomb.at[pl.ds(base, NR)], iss[isl], semi[isl])
        for j in range(NR):
            pltpu.async_copy(
                packed.at[igc[p].at[j]], rows0[p].at[pl.ds(j * 128, 128)], semg[p])
            pltpu.async_copy(
                packed.at[igc[p].at[NR + j]], rows1[p].at[pl.ds(j * 128, 128)],
                semg[p])

    def drain_gathers(p):
        for j in range(2 * NR):
            pltpu.make_async_copy(
                packed.at[igc[p].at[0]], rows0[p].at[pl.ds(0, 128)],
                semg[p]).wait()

    def drain_scatters(p):
        for j in range(2 * NR):
            pltpu.make_async_copy(
                z8.at[pl.ds(0, 128)], acc_a.at[pl.ds(0, 128)], sems[p]).wait()

    def edge_compute(p):
        @pl.loop(0, CE // 16)
        def _grp(g):
            rid = g * 16 + iota16
            ld0 = lambda k: plsc.load_gather(rows0[p], [rid, col(k)])
            ld1 = lambda k: plsc.load_gather(rows1[p], [rid, col(k)])
            s0 = [ld0(k) for k in range(3)]
            s1 = [ld1(k) for k in range(3)]
            v1 = [ld1(3 + k) for k in range(3)]
            p0 = [ld0(6 + k) for k in range(3)]
            p1 = [ld1(6 + k) for k in range(3)]
            h = s0 + s1
            hid = []
            for j in range(3):
                a = B1S(j)
                for k in range(6):
                    a = a + h[k] * W1S(k, j)
                hid.append(silu(a))
            gate = []
            for m in range(9):
                a = B2S(m)
                for j in range(3):
                    a = a + hid[j] * W2S(j, m)
                gate.append(a)
            for d in range(3):
                plsc.store_scatter(ea[p], [rid, col(d)], gate[6 + d])
                plsc.store_scatter(ea[p], [rid, col(3 + d)], gate[d] * v1[d])
                plsc.store_scatter(eb[p], [rid, col(d)], gate[3 + d] * (p1[d] - p0[d]))

    def edge_step(ck, p, q, isl, isl_next):
        @pl.when(ck + 1 < n_ch_e)
        def _():
            fire_gathers(ck + 1, q, isl_next)
        drain_gathers(p)

        @pl.when(ck >= 2)
        def _():
            drain_scatters(p)
        edge_compute(p)
        pltpu.make_async_copy(
            ecomb.at[pl.ds(0, NR)], iss[isl], semi[isl]).wait()
        for j in range(NR):
            pltpu.async_copy(
                ea[p].at[pl.ds(j * 128, 128)], acc_a.at[iss[isl].at[j]],
                sems[p], add=True)
            pltpu.async_copy(
                eb[p].at[pl.ds(j * 128, 128)], acc_b.at[iss[isl].at[j]],
                sems[p], add=True)

    fire_gathers(0, 0, 0)

    @pl.loop(0, n_ch_e // 4)
    def _edge_iter(k4):
        for u in range(4):
            edge_step(k4 * 4 + u, u % 2, 1 - u % 2, u, (u + 1) % 4)

    drain_scatters(0)
    drain_scatters(1)

    pltpu.sync_copy(z8.at[pl.ds(0, CE)], ea[0])
    pltpu.sync_copy(z8.at[pl.ds(0, CE)], ea[1])

    def tri_fire(ck, p, isl):
        base = (w * n_ch_t + ck) * 4
        pltpu.sync_copy(tcomb.at[pl.ds(base, 4)], igc[p])
        pltpu.async_copy(tcomb.at[pl.ds(base, 1)], iss[isl].at[pl.ds(0, 1)],
                         semi[isl])
        pltpu.async_copy(packed.at[igc[p].at[0]],
                         rows0[p].at[pl.ds(0, 128)], semg[p])
        pltpu.async_copy(packed.at[igc[p].at[1]],
                         rows0[p].at[pl.ds(128, 128)], semg[p])
        pltpu.async_copy(packed.at[igc[p].at[2]],
                         rows1[p].at[pl.ds(0, 128)], semg[p])
        pltpu.async_copy(packed.at[igc[p].at[3]],
                         rows1[p].at[pl.ds(128, 128)], semg[p])

    def tri_drain_gathers(p):
        for j in range(4):
            pltpu.make_async_copy(
                packed.at[igc[p].at[0]], rows0[p].at[pl.ds(0, 128)],
                semg[p]).wait()

    def tri_drain_scatters(p):
        pltpu.make_async_copy(
            z8.at[pl.ds(0, 128)], acc_b.at[pl.ds(0, 128)], sems[p]).wait()

    def tri_compute(p):
        @pl.loop(0, CT // 16)
        def _grp(g):
            rid = g * 16 + iota16
            ldb = lambda k: plsc.load_gather(rows0[p], [rid, col(k)])
            ld1 = lambda k: plsc.load_gather(rows0[p], [rid + CT, col(k)])
            ld2 = lambda k: plsc.load_gather(rows1[p], [rid, col(k)])
            ld3 = lambda k: plsc.load_gather(rows1[p], [rid + CT, col(k)])
            cb = [ldb(9 + k) for k in range(3)]
            c1 = [ld1(9 + k) for k in range(3)]
            c2 = [ld2(9 + k) for k in range(3)]
            c3 = [ld3(9 + k) for k in range(3)]
            pb = [ldb(6 + k) for k in range(3)]
            q1 = [ld1(6 + k) for k in range(3)]
            q2 = [ld2(6 + k) for k in range(3)]
            q3 = [ld3(6 + k) for k in range(3)]

            gate = [B2C(m) * 3.0 for m in range(3)]
            for ct in (c1, c2, c3):
                hid = []
                for j in range(3):
                    a = B1C(j)
                    for k in range(3):
                        a = a + cb[k] * W1C(k, j)
                    for k in range(3):
                        a = a + ct[k] * W1C(3 + k, j)
                    hid.append(silu(a))
                for m in range(3):
                    acc = gate[m]
                    for j in range(3):
                        acc = acc + hid[j] * W2C(j, m)
                    gate[m] = acc

            r1 = [pb[k] - q1[k] for k in range(3)]
            r2 = [pb[k] - q2[k] for k in range(3)]
            r3 = [pb[k] - q3[k] for k in range(3)]
            cx = r2[1] * r3[2] - r2[2] * r3[1]
            cy = r2[2] * r3[0] - r2[0] * r3[2]
            cz = r2[0] * r3[1] - r2[1] * r3[0]
            stp = r1[0] * cx + r1[1] * cy + r1[2] * cz
            inv = 1.0 / (stp + 0.01)
            for m in range(3):
                plsc.store_scatter(ea[p], [rid, col(3 + m)], gate[m] * inv)

    def tri_step(ck, p, q, isl, isl_next):
        @pl.when(ck + 1 < n_ch_t)
        def _():
            tri_fire(ck + 1, q, isl_next)
        tri_drain_gathers(p)

        @pl.when(ck >= 2)
        def _():
            tri_drain_scatters(p)
        tri_compute(p)
        pltpu.make_async_copy(
            tcomb.at[pl.ds(0, 1)], iss[isl].at[pl.ds(0, 1)], semi[isl]).wait()
        pltpu.async_copy(ea[p].at[pl.ds(0, 128)], acc_b.at[iss[isl].at[0]],
                         sems[p], add=True)

    tri_fire(0, 0, 0)

    @pl.loop(0, n_ch_t // 4)
    def _tri_iter(k4):
        for u in range(4):
            tri_step(k4 * 4 + u, u % 2, 1 - u % 2, u, (u + 1) % 4)

    tri_drain_scatters(0)
    tri_drain_scatters(1)

    plsc.subcore_barrier()
    pltpu.sync_copy(acc_a.at[pl.ds(r0, RT)], outa.at[pl.ds(c * NACC + r0, RT)])
    pltpu.sync_copy(acc_b.at[pl.ds(r0, RT)], outb.at[pl.ds(c * NACC + r0, RT)])


def _tc_body(wv_ref, bv_ref, s_ref, c_ref, v_ref, ss_ref, vv_ref, ev_ref, ch_ref,
             os_ref, oc_ref, ov_ref):
    sc = s_ref[...]
    ch = c_ref[...]
    ve = v_ref[...]
    m_ss = ss_ref[0] + ss_ref[1]
    m_vv = vv_ref[0] + vv_ref[1]
    m_ev = ev_ref[0] + ev_ref[1]
    m_ch = ch_ref[0] + ch_ref[1]
    norm2 = jnp.zeros((BN, 1), _f32)
    for j in range(3):
        vvj = (bv_ref[j] + ve[:, 0:1] * wv_ref[0, j] + ve[:, 1:2] * wv_ref[1, j]
               + ve[:, 2:3] * wv_ref[2, j])
        norm2 = norm2 + vvj * vvj
    nrm = jnp.sqrt(norm2)
    os_ref[...] = sc + m_ss + sc * nrm
    oc_ref[...] = ch + m_ch
    ov_ref[...] = ve + m_vv + m_ev


def kernel(node_scalar, node_chiral, node_vector, edge_index, triplet_index, pos,
           W1s, b1s, W2s, b2s, W1c, b1c, W2c, b2c, WV, bV):
    E = edge_index.shape[0]
    T = triplet_index.shape[0]
    n_ch_e = -(-E // (NW * CE))
    n_ch_e += (-n_ch_e) % 4
    n_ch_t = -(-T // (NW * CT))
    n_ch_t += (-n_ch_t) % 4
    EPAD = NW * CE * n_ch_e
    TPAD = NW * CT * n_ch_t

    zcol = jnp.zeros((NACC - N_NODES, 3), _f32)
    packed = jnp.concatenate([
        jnp.concatenate([node_scalar, zcol], 0),
        jnp.concatenate([node_vector, zcol], 0),
        jnp.concatenate([pos, zcol], 0),
        jnp.concatenate([node_chiral, zcol], 0),
        jnp.zeros((NACC, 4), _f32),
    ], axis=1)

    def pad_idx(x, total, nr):
        npad = total - x.shape[0]
        fill = N_NODES + (jnp.arange(npad, dtype=_i32) % NDUM)
        return jnp.concatenate([x.astype(_i32), fill]).reshape(-1, nr, 128)

    ecomb = jnp.concatenate([
        pad_idx(edge_index[:, 0], EPAD, NR),
        pad_idx(edge_index[:, 1], EPAD, NR),
    ], axis=1).reshape(-1, 128)
    tcomb = jnp.concatenate([
        pad_idx(triplet_index[:, 0], TPAD, NRT),
        pad_idx(triplet_index[:, 1], TPAD, NRT),
        pad_idx(triplet_index[:, 2], TPAD, NRT),
        pad_idx(triplet_index[:, 3], TPAD, NRT),
    ], axis=1).reshape(-1, 128)

    wpk = jnp.concatenate([
        W1s.reshape(-1), b1s, W2s.reshape(-1), b2s,
        W1c.reshape(-1), b1c, W2c.reshape(-1), b2c,
    ]).astype(_f32)[:, None] * jnp.ones((1, 16), _f32)

    z8 = jnp.zeros((NACC, 8), _f32)

    mesh = plsc.VectorSubcoreMesh(core_axis_name="c", subcore_axis_name="s")
    sc_fn = functools.partial(
        pl.kernel,
        out_type=(jax.ShapeDtypeStruct((NC * NACC, 8), _f32),
                  jax.ShapeDtypeStruct((NC * NACC, 8), _f32)),
        mesh=mesh,
        compiler_params=pltpu.CompilerParams(
            needs_layout_passes=False, use_tc_tiling_on_sc=False),
        scratch_types=[
            pltpu.VMEM((90, 16), _f32),
            pltpu.VMEM((2 * NR, 128), _i32),
            pltpu.VMEM((2 * NR, 128), _i32),
            pltpu.VMEM((NR, 128), _i32),
            pltpu.VMEM((NR, 128), _i32),
            pltpu.VMEM((NR, 128), _i32),
            pltpu.VMEM((NR, 128), _i32),
            pltpu.VMEM((CE, 16), _f32),
            pltpu.VMEM((CE, 16), _f32),
            pltpu.VMEM((CE, 16), _f32),
            pltpu.VMEM((CE, 16), _f32),
            pltpu.VMEM((CE, 8), _f32),
            pltpu.VMEM((CE, 8), _f32),
            pltpu.VMEM((CE, 8), _f32),
            pltpu.VMEM((CE, 8), _f32),
            pltpu.VMEM_SHARED((NACC, 8), _f32),
            pltpu.VMEM_SHARED((NACC, 8), _f32),
            pltpu.SemaphoreType.DMA,
            pltpu.SemaphoreType.DMA,
            pltpu.SemaphoreType.DMA,
            pltpu.SemaphoreType.DMA,
            pltpu.SemaphoreType.DMA,
            pltpu.SemaphoreType.DMA,
            pltpu.SemaphoreType.DMA,
            pltpu.SemaphoreType.DMA,
        ],
    )(functools.partial(_sc_body, n_ch_e=n_ch_e, n_ch_t=n_ch_t))

    outa, outb = sc_fn(packed, ecomb, tcomb, wpk, z8)

    ea = outa.reshape(NC, NACC, 8)
    eb = outb.reshape(NC, NACC, 8)

    def prep(x):
        return jnp.pad(x[:, :N_NODES, :], ((0, 0), (0, NP - N_NODES), (0, 0)))

    ssP = prep(ea[:, :, 0:3])
    vvP = prep(ea[:, :, 3:6])
    evP = prep(eb[:, :, 0:3])
    chP = prep(eb[:, :, 3:6])

    def padn(x):
        return jnp.pad(x, ((0, NP - N_NODES), (0, 0)))

    sP = padn(node_scalar)
    cP = padn(node_chiral)
    vP = padn(node_vector)

    nblk = pl.BlockSpec((BN, 3), lambda i: (i, 0))
    pblk = pl.BlockSpec((NC, BN, 3), lambda i: (0, i, 0))
    smem = pl.BlockSpec(memory_space=pltpu.SMEM)
    outs = pl.pallas_call(
        _tc_body,
        grid=(NP // BN,),
        in_specs=[smem, smem, nblk, nblk, nblk, pblk, pblk, pblk, pblk],
        out_specs=[nblk, nblk, nblk],
        out_shape=[jax.ShapeDtypeStruct((NP, 3), _f32)] * 3,
    )(WV, bV, sP, cP, vP, ssP, vvP, evP, chP)

    return (outs[0][:N_NODES], outs[1][:N_NODES], outs[2][:N_NODES])
